# Initial kernel scaffold; baseline (speedup 1.0000x reference)
import jax
import jax.numpy as jnp
from jax.experimental import pallas as pl


def kernel(x, pos, edge_index, W, b):
    raise NotImplementedError("write your pallas kernel here")


# speedup vs baseline: 9.3644x; 9.3644x over previous
import functools

import jax
import jax.numpy as jnp
from jax import lax
from jax.experimental import pallas as pl
from jax.experimental.pallas import tpu as pltpu
from jax.experimental.pallas import tpu_sc as plsc

N = 10000
E = 320000
D = 128
DA = 144
NW = 32
EPT = E // NW
CHUNK = 128
NFULL = EPT // CHUNK
TAIL = EPT - NFULL * CHUNK
NP = 10240
RPT = NP // 16
BLK = 1000


def _prep_body(x_ref, ps_ref, pc_ref, wx_ref, wp_ref, b_ref, ua_ref, t_ref):
    u = jnp.dot(x_ref[...], wx_ref[...], preferred_element_type=jnp.float32)
    wp = wp_ref[...]
    for k in range(3):
        u = u + ps_ref[:, k : k + 1] * wp[k : k + 1, :]
    tcol = b_ref[...]
    for k in range(3):
        tcol = tcol - pc_ref[:, k : k + 1] * wp[k : k + 1, :]
    lane = lax.broadcasted_iota(jnp.int32, (BLK, DA - D), 1)
    pad = jnp.where(lane == 0, 1.0, 0.0).astype(jnp.float32)
    ua_ref[...] = jnp.concatenate([u, pad], axis=1)
    t_ref[...] = tcol


def _prep(x_src, pos_src, pos_c, wx, wp, b2):
    return pl.pallas_call(
        _prep_body,
        grid=(N // BLK,),
        in_specs=[
            pl.BlockSpec((BLK, D), lambda i: (i, 0)),
            pl.BlockSpec((BLK, 8), lambda i: (i, 0)),
            pl.BlockSpec((BLK, 8), lambda i: (i, 0)),
            pl.BlockSpec((D, D), lambda i: (0, 0)),
            pl.BlockSpec((8, D), lambda i: (0, 0)),
            pl.BlockSpec((1, D), lambda i: (0, 0)),
        ],
        out_specs=[
            pl.BlockSpec((BLK, DA), lambda i: (i, 0)),
            pl.BlockSpec((BLK, D), lambda i: (i, 0)),
        ],
        out_shape=[
            jax.ShapeDtypeStruct((N, DA), jnp.float32),
            jax.ShapeDtypeStruct((N, D), jnp.float32),
        ],
    )(x_src, pos_src, pos_c, wx, wp, b2)


def _sc_body(u_hbm, src_hbm, dst_hbm, z_hbm, out0_hbm, out1_hbm,
             src_v, dst_v, buf, src_t, dst_t, buf_t, acc, sem):
    c = lax.axis_index("c")
    s = lax.axis_index("s")
    wid = c * 16 + s

    pltpu.sync_copy(z_hbm.at[pl.ds(s * RPT, RPT)], acc.at[pl.ds(s * RPT, RPT)])
    plsc.subcore_barrier()

    ebase = wid * EPT

    def chunk(i, carry):
        off = pl.multiple_of(ebase + i * CHUNK, 8)
        pltpu.sync_copy(src_hbm.at[pl.ds(off, CHUNK)], src_v)
        pltpu.sync_copy(dst_hbm.at[pl.ds(off, CHUNK)], dst_v)
        pltpu.async_copy(u_hbm.at[src_v], buf, sem).wait()
        pltpu.sync_copy(buf, acc.at[dst_v], add=True)
        return carry

    lax.fori_loop(0, NFULL, chunk, 0)

    toff = ebase + NFULL * CHUNK
    pltpu.sync_copy(src_hbm.at[pl.ds(toff, TAIL)], src_t)
    pltpu.sync_copy(dst_hbm.at[pl.ds(toff, TAIL)], dst_t)
    pltpu.async_copy(u_hbm.at[src_t], buf_t, sem).wait()
    pltpu.sync_copy(buf_t, acc.at[dst_t], add=True)

    plsc.subcore_barrier()

    @pl.when(c == 0)
    def _():
        pltpu.sync_copy(acc.at[pl.ds(s * RPT, RPT)], out0_hbm.at[pl.ds(s * RPT, RPT)])

    @pl.when(c == 1)
    def _():
        pltpu.sync_copy(acc.at[pl.ds(s * RPT, RPT)], out1_hbm.at[pl.ds(s * RPT, RPT)])


def _scatter(u_aug, src, dst, zeros):
    mesh = plsc.VectorSubcoreMesh(
        core_axis_name="c", subcore_axis_name="s", num_cores=2, num_subcores=16
    )
    f = pl.kernel(
        _sc_body,
        mesh=mesh,
        out_type=[
            jax.ShapeDtypeStruct((NP, DA), jnp.float32),
            jax.ShapeDtypeStruct((NP, DA), jnp.float32),
        ],
        scratch_types=[
            pltpu.VMEM((CHUNK,), jnp.int32),
            pltpu.VMEM((CHUNK,), jnp.int32),
            pltpu.VMEM((CHUNK, DA), jnp.float32),
            pltpu.VMEM((TAIL,), jnp.int32),
            pltpu.VMEM((TAIL,), jnp.int32),
            pltpu.VMEM((TAIL, DA), jnp.float32),
            pltpu.VMEM_SHARED((NP, DA), jnp.float32),
            pltpu.SemaphoreType.DMA,
        ],
        compiler_params=pltpu.CompilerParams(use_tc_tiling_on_sc=False),
    )
    return f(u_aug, src, dst, zeros)


def _combine_body(p0_ref, p1_ref, t_ref, o_ref):
    p0 = p0_ref[...]
    p1 = p1_ref[...]
    s = p0[:, :D] + p1[:, :D]
    cnt = p0[:, D : D + 1] + p1[:, D : D + 1]
    o_ref[...] = s + cnt * t_ref[...]


def _combine(p0, p1, t):
    return pl.pallas_call(
        _combine_body,
        grid=(N // BLK,),
        in_specs=[
            pl.BlockSpec((BLK, DA), lambda i: (i, 0)),
            pl.BlockSpec((BLK, DA), lambda i: (i, 0)),
            pl.BlockSpec((BLK, D), lambda i: (i, 0)),
        ],
        out_specs=pl.BlockSpec((BLK, D), lambda i: (i, 0)),
        out_shape=jax.ShapeDtypeStruct((N, D), jnp.float32),
    )(p0, p1, t)


@jax.jit
def kernel(x, pos, edge_index, W, b):
    x_src = x[0]
    pos_src = pos[0]
    pos_c = pos[1]
    src = edge_index[0]
    dst = edge_index[1]

    wx = W[:D]
    wp = jnp.pad(W[D:], ((0, 5), (0, 0)))
    b2 = b[None, :]
    ps = jnp.pad(pos_src, ((0, 0), (0, 5)))
    pc = jnp.pad(pos_c, ((0, 0), (0, 5)))

    u_aug, t = _prep(x_src, ps, pc, wx, wp, b2)
    zeros = jnp.zeros((NP, DA), jnp.float32)
    p0, p1 = _scatter(u_aug, src, dst, zeros)
    return _combine(p0, p1, t)


# speedup vs baseline: 12.4123x; 1.3255x over previous
import functools

import jax
import jax.numpy as jnp
from jax import lax
from jax.experimental import pallas as pl
from jax.experimental.pallas import tpu as pltpu
from jax.experimental.pallas import tpu_sc as plsc

N = 10000
E = 320000
D = 128
DA = 144
NW = 32
EPT = E // NW
CHUNK = 100
NCH = EPT // CHUNK
PH = 20
NPH = NCH // PH
NP = 10240
RPT = NP // 16
BLK = 1000


def _prep_body(x_ref, ps_ref, pc_ref, wx_ref, wp_ref, b_ref, ua_ref, t_ref):
    u = jnp.dot(x_ref[...], wx_ref[...], preferred_element_type=jnp.float32)
    wp = wp_ref[...]
    for k in range(3):
        u = u + ps_ref[:, k : k + 1] * wp[k : k + 1, :]
    tcol = b_ref[...]
    for k in range(3):
        tcol = tcol - pc_ref[:, k : k + 1] * wp[k : k + 1, :]
    lane = lax.broadcasted_iota(jnp.int32, (BLK, DA - D), 1)
    pad = jnp.where(lane == 0, 1.0, 0.0).astype(jnp.float32)
    ua_ref[...] = jnp.concatenate([u, pad], axis=1)
    t_ref[...] = tcol


def _prep(x_src, pos_src, pos_c, wx, wp, b2):
    return pl.pallas_call(
        _prep_body,
        grid=(N // BLK,),
        in_specs=[
            pl.BlockSpec((BLK, D), lambda i: (i, 0)),
            pl.BlockSpec((BLK, 8), lambda i: (i, 0)),
            pl.BlockSpec((BLK, 8), lambda i: (i, 0)),
            pl.BlockSpec((D, D), lambda i: (0, 0)),
            pl.BlockSpec((8, D), lambda i: (0, 0)),
            pl.BlockSpec((1, D), lambda i: (0, 0)),
        ],
        out_specs=[
            pl.BlockSpec((BLK, DA), lambda i: (i, 0)),
            pl.BlockSpec((BLK, D), lambda i: (i, 0)),
        ],
        out_shape=[
            jax.ShapeDtypeStruct((N, DA), jnp.float32),
            jax.ShapeDtypeStruct((N, D), jnp.float32),
        ],
    )(x_src, pos_src, pos_c, wx, wp, b2)


def _sc_body(u_hbm, src_hbm, dst_hbm, z_hbm, out0_hbm, out1_hbm,
             src_v, dst_v, buf0, buf1, acc, gsem, ssem):
    c = lax.axis_index("c")
    s = lax.axis_index("s")
    wid = c * 16 + s

    pltpu.sync_copy(z_hbm.at[pl.ds(s * RPT, RPT)], acc.at[pl.ds(s * RPT, RPT)])
    plsc.subcore_barrier()

    bufs = (buf0, buf1)

    def gather(j, b):
        pltpu.async_copy(u_hbm.at[src_v.at[j]], b, gsem)

    def gwait(b):
        pltpu.make_async_copy(u_hbm.at[src_v.at[0]], b, gsem).wait()

    def scat(j, b):
        pltpu.async_copy(b, acc.at[dst_v.at[j]], ssem, add=True)

    def swait(b):
        pltpu.make_async_copy(b, acc.at[dst_v.at[0]], ssem).wait()

    def step(i, carry):
        for p in range(2):
            j = 2 * i + p
            b, ob = bufs[p], bufs[1 - p]
            gwait(b)
            if p == 0:
                @pl.when(i > 0)
                def _():
                    swait(ob)
                gather(j + 1, ob)
            else:
                swait(ob)

                @pl.when(i < PH // 2 - 1)
                def _():
                    gather(j + 1, ob)
            scat(j, b)
        return carry

    def phase(ph, carry):
        base = wid * NCH + ph * PH
        pltpu.sync_copy(src_hbm.at[pl.ds(base, PH)], src_v)
        pltpu.sync_copy(dst_hbm.at[pl.ds(base, PH)], dst_v)
        gather(0, buf0)
        lax.fori_loop(0, PH // 2, step, 0)
        swait(buf1)
        return carry

    lax.fori_loop(0, NPH, phase, 0)

    plsc.subcore_barrier()

    @pl.when(c == 0)
    def _():
        pltpu.sync_copy(acc.at[pl.ds(s * RPT, RPT)], out0_hbm.at[pl.ds(s * RPT, RPT)])

    @pl.when(c == 1)
    def _():
        pltpu.sync_copy(acc.at[pl.ds(s * RPT, RPT)], out1_hbm.at[pl.ds(s * RPT, RPT)])


def _scatter(u_aug, src, dst, zeros):
    mesh = plsc.VectorSubcoreMesh(
        core_axis_name="c", subcore_axis_name="s", num_cores=2, num_subcores=16
    )
    f = pl.kernel(
        _sc_body,
        mesh=mesh,
        out_type=[
            jax.ShapeDtypeStruct((NP, DA), jnp.float32),
            jax.ShapeDtypeStruct((NP, DA), jnp.float32),
        ],
        scratch_types=[
            pltpu.VMEM((PH, CHUNK), jnp.int32),
            pltpu.VMEM((PH, CHUNK), jnp.int32),
            pltpu.VMEM((CHUNK, DA), jnp.float32),
            pltpu.VMEM((CHUNK, DA), jnp.float32),
            pltpu.VMEM_SHARED((NP, DA), jnp.float32),
            pltpu.SemaphoreType.DMA,
            pltpu.SemaphoreType.DMA,
        ],
        compiler_params=pltpu.CompilerParams(use_tc_tiling_on_sc=False),
    )
    return f(u_aug, src, dst, zeros)


def _combine_body(p0_ref, p1_ref, t_ref, o_ref):
    p0 = p0_ref[...]
    p1 = p1_ref[...]
    s = p0[:, :D] + p1[:, :D]
    cnt = p0[:, D : D + 1] + p1[:, D : D + 1]
    o_ref[...] = s + cnt * t_ref[...]


def _combine(p0, p1, t):
    return pl.pallas_call(
        _combine_body,
        grid=(N // BLK,),
        in_specs=[
            pl.BlockSpec((BLK, DA), lambda i: (i, 0)),
            pl.BlockSpec((BLK, DA), lambda i: (i, 0)),
            pl.BlockSpec((BLK, D), lambda i: (i, 0)),
        ],
        out_specs=pl.BlockSpec((BLK, D), lambda i: (i, 0)),
        out_shape=jax.ShapeDtypeStruct((N, D), jnp.float32),
    )(p0, p1, t)


@jax.jit
def kernel(x, pos, edge_index, W, b):
    x_src = x[0]
    pos_src = pos[0]
    pos_c = pos[1]
    src = edge_index[0]
    dst = edge_index[1]

    wx = W[:D]
    wp = jnp.pad(W[D:], ((0, 5), (0, 0)))
    b2 = b[None, :]
    ps = jnp.pad(pos_src, ((0, 0), (0, 5)))
    pc = jnp.pad(pos_c, ((0, 0), (0, 5)))

    u_aug, t = _prep(x_src, ps, pc, wx, wp, b2)
    zeros = jnp.zeros((NP, DA), jnp.float32)
    src2 = src.reshape(E // CHUNK, CHUNK)
    dst2 = dst.reshape(E // CHUNK, CHUNK)
    p0, p1 = _scatter(u_aug, src2, dst2, zeros)
    return _combine(p0, p1, t)
